# Initial kernel scaffold; baseline (speedup 1.0000x reference)
import jax
import jax.numpy as jnp
from jax.experimental import pallas as pl


def kernel(x, noise):
    raise NotImplementedError("write your pallas kernel here")


# speedup vs baseline: 7.6583x; 7.6583x over previous
import functools

import jax
import jax.numpy as jnp
from jax import lax
from jax.experimental import pallas as pl
from jax.experimental.pallas import tpu as pltpu
from jax.experimental.pallas import tpu_sc as plsc

B = 4
T = 8192
C = 768
KEEP = 4096
ROWS = 64
LANES = 128
OUT_ROWS = KEEP // LANES


def _sort_body(keys_ref, out_ref):
    vf = keys_ref[...]
    u = lax.bitcast_convert_type(vf, jnp.int32)
    v = u ^ (lax.shift_right_arithmetic(u, 31) & jnp.int32(0x7FFFFFFF))
    r = lax.broadcasted_iota(jnp.int32, (B, ROWS, LANES), 1)
    c = lax.broadcasted_iota(jnp.int32, (B, ROWS, LANES), 2)
    pos = r * LANES + c
    payload = pos

    for kbit in range(1, 14):
        k = 1 << kbit
        for sbit in range(kbit - 1, -1, -1):
            s = 1 << sbit
            ax, sh = (2, s) if s < LANES else (1, s // LANES)
            bit_set = (pos & s) != 0
            pv = jnp.where(bit_set, jnp.roll(v, sh, axis=ax),
                           jnp.roll(v, -sh, axis=ax))
            pi = jnp.where(bit_set, jnp.roll(payload, sh, axis=ax),
                           jnp.roll(payload, -sh, axis=ax))
            p_precedes = (pv > v) | ((pv == v) & (pi < payload))
            ascending = (pos & k) == 0
            want_preceding = (~bit_set) == ascending
            take_partner = want_preceding == p_precedes
            v = jnp.where(take_partner, pv, v)
            payload = jnp.where(take_partner, pi, payload)

    boff = lax.broadcasted_iota(jnp.int32, (B, OUT_ROWS, LANES), 0) * T
    out_ref[...] = payload[:, :OUT_ROWS, :] + boff


_sort_call = pl.pallas_call(
    _sort_body,
    out_shape=jax.ShapeDtypeStruct((B, OUT_ROWS, LANES), jnp.int32),
)


_NW = 32
_N_IDX = B * KEEP
_PER_W = _N_IDX // _NW
_CHUNK = 64
_NCHUNK = _PER_W // _CHUNK


@functools.cache
def _make_gather():
    def body(table_hbm, idx_hbm, out_hbm, idx_v, rows_v, sem):
        wid = lax.axis_index("s") * 2 + lax.axis_index("c")
        base = wid * _PER_W
        for kk in range(_NCHUNK):
            ck = base + kk * _CHUNK
            pltpu.sync_copy(idx_hbm.at[pl.ds(ck, _CHUNK)], idx_v)
            pltpu.async_copy(table_hbm.at[idx_v], rows_v, sem).wait()
            pltpu.sync_copy(rows_v, out_hbm.at[pl.ds(ck, _CHUNK)])

    return pl.kernel(
        body,
        mesh=plsc.VectorSubcoreMesh(core_axis_name="c", subcore_axis_name="s"),
        out_type=jax.ShapeDtypeStruct((_N_IDX, C), jnp.float32),
        scratch_types=[
            pltpu.VMEM((_CHUNK,), jnp.int32),
            pltpu.VMEM((_CHUNK, C), jnp.float32),
            pltpu.SemaphoreType.DMA,
        ],
    )


def kernel(x, noise):
    keys = jnp.concatenate(
        [jnp.full((B, 1), jnp.inf, noise.dtype), noise], axis=1)
    gidx = _sort_call(keys.reshape(B, ROWS, LANES))
    table = x.reshape(B * T, C)
    out = _make_gather()(table, gidx.reshape(_N_IDX))
    return out.reshape(B, KEEP, C)


# speedup vs baseline: 8.1169x; 1.0599x over previous
import functools

import jax
import jax.numpy as jnp
from jax import lax
from jax.experimental import pallas as pl
from jax.experimental.pallas import tpu as pltpu
from jax.experimental.pallas import tpu_sc as plsc

B = 4
T = 8192
C = 768
KEEP = 4096
ROWS = 64
LANES = 128
OUT_ROWS = KEEP // LANES


def _sort_body(keys_ref, out_ref):
    vf = keys_ref[...]
    u = lax.bitcast_convert_type(vf, jnp.int32)
    v = u ^ (lax.shift_right_arithmetic(u, 31) & jnp.int32(0x7FFFFFFF))
    r = lax.broadcasted_iota(jnp.int32, (B, ROWS, LANES), 1)
    c = lax.broadcasted_iota(jnp.int32, (B, ROWS, LANES), 2)
    pos = r * LANES + c
    payload = pos

    for kbit in range(1, 14):
        k = 1 << kbit
        for sbit in range(kbit - 1, -1, -1):
            s = 1 << sbit
            ax, sh = (2, s) if s < LANES else (1, s // LANES)
            bit_set = (pos & s) != 0
            pv = jnp.where(bit_set, jnp.roll(v, sh, axis=ax),
                           jnp.roll(v, -sh, axis=ax))
            pi = jnp.where(bit_set, jnp.roll(payload, sh, axis=ax),
                           jnp.roll(payload, -sh, axis=ax))
            p_precedes = (pv > v) | ((pv == v) & (pi < payload))
            ascending = (pos & k) == 0
            want_preceding = (~bit_set) == ascending
            take_partner = want_preceding == p_precedes
            v = jnp.where(take_partner, pv, v)
            payload = jnp.where(take_partner, pi, payload)

    boff = lax.broadcasted_iota(jnp.int32, (B, OUT_ROWS, LANES), 0) * T
    out_ref[...] = payload[:, :OUT_ROWS, :] + boff


_sort_call = pl.pallas_call(
    _sort_body,
    out_shape=jax.ShapeDtypeStruct((B, OUT_ROWS, LANES), jnp.int32),
)


_NW = 32
_N_IDX = B * KEEP
_PER_W = _N_IDX // _NW
_CHUNK = 64
_NCHUNK = _PER_W // _CHUNK


@functools.cache
def _make_gather():
    def body(table_hbm, idx_hbm, out_hbm, idx_v, rows0, rows1, gs0, gs1,
             os0, os1):
        wid = lax.axis_index("s") * 2 + lax.axis_index("c")
        base = wid * _PER_W
        pltpu.sync_copy(idx_hbm.at[pl.ds(base, _PER_W)], idx_v)
        rows = (rows0, rows1)
        gsem = (gs0, gs1)
        osem = (os0, os1)
        gth = [None, None]
        oth = [None, None]
        gth[0] = pltpu.async_copy(
            table_hbm.at[idx_v.at[pl.ds(0, _CHUNK)]], rows0, gs0)
        for kk in range(_NCHUNK):
            cur, nxt = kk % 2, (kk + 1) % 2
            gth[cur].wait()
            if kk >= 1:
                oth[nxt].wait()
            if kk + 1 < _NCHUNK:
                gth[nxt] = pltpu.async_copy(
                    table_hbm.at[idx_v.at[pl.ds((kk + 1) * _CHUNK, _CHUNK)]],
                    rows[nxt], gsem[nxt])
            oth[cur] = pltpu.async_copy(
                rows[cur], out_hbm.at[pl.ds(base + kk * _CHUNK, _CHUNK)],
                osem[cur])
        oth[(_NCHUNK - 1) % 2].wait()

    return pl.kernel(
        body,
        mesh=plsc.VectorSubcoreMesh(core_axis_name="c", subcore_axis_name="s"),
        out_type=jax.ShapeDtypeStruct((_N_IDX, C), jnp.float32),
        scratch_types=[
            pltpu.VMEM((_PER_W,), jnp.int32),
            pltpu.VMEM((_CHUNK, C), jnp.float32),
            pltpu.VMEM((_CHUNK, C), jnp.float32),
            pltpu.SemaphoreType.DMA,
            pltpu.SemaphoreType.DMA,
            pltpu.SemaphoreType.DMA,
            pltpu.SemaphoreType.DMA,
        ],
    )


def kernel(x, noise):
    keys = jnp.concatenate(
        [jnp.full((B, 1), jnp.inf, noise.dtype), noise], axis=1)
    gidx = _sort_call(keys.reshape(B, ROWS, LANES))
    table = x.reshape(B * T, C)
    out = _make_gather()(table, gidx.reshape(_N_IDX))
    return out.reshape(B, KEEP, C)


# speedup vs baseline: 8.2800x; 1.0201x over previous
import functools

import jax
import jax.numpy as jnp
from jax import lax
from jax.experimental import pallas as pl
from jax.experimental.pallas import tpu as pltpu
from jax.experimental.pallas import tpu_sc as plsc

B = 4
T = 8192
C = 768
KEEP = 4096
ROWS = 64
GB = 4
LANES = 128
OUT_ROWS = KEEP // LANES


def _sort_body(keys_ref, out_ref):
    vf = keys_ref[...]
    u = lax.bitcast_convert_type(vf, jnp.int32)
    v = u ^ (lax.shift_right_arithmetic(u, 31) & jnp.int32(0x7FFFFFFF))
    r = lax.broadcasted_iota(jnp.int32, (GB, ROWS, LANES), 1)
    c = lax.broadcasted_iota(jnp.int32, (GB, ROWS, LANES), 2)
    pos = r * LANES + c
    payload = pos

    for kbit in range(1, 14):
        k = 1 << kbit
        for sbit in range(kbit - 1, -1, -1):
            s = 1 << sbit
            ax, sh = (2, s) if s < LANES else (1, s // LANES)
            bit_set = (pos & s) != 0
            pv = jnp.where(bit_set, jnp.roll(v, sh, axis=ax),
                           jnp.roll(v, -sh, axis=ax))
            pi = jnp.where(bit_set, jnp.roll(payload, sh, axis=ax),
                           jnp.roll(payload, -sh, axis=ax))
            p_follows = (pv < v) | ((pv == v) & (pi > payload))
            ascending = (pos & k) == 0
            take_partner = bit_set ^ ascending ^ p_follows
            v = jnp.where(take_partner, pv, v)
            payload = jnp.where(take_partner, pi, payload)

    boff = (pl.program_id(0) * GB
            + lax.broadcasted_iota(jnp.int32, (GB, OUT_ROWS, LANES), 0)) * T
    out_ref[...] = payload[:, :OUT_ROWS, :] + boff


_sort_call = pl.pallas_call(
    _sort_body,
    grid=(B // GB,),
    in_specs=[pl.BlockSpec((GB, ROWS, LANES), lambda b: (b, 0, 0))],
    out_specs=pl.BlockSpec((GB, OUT_ROWS, LANES), lambda b: (b, 0, 0)),
    out_shape=jax.ShapeDtypeStruct((B, OUT_ROWS, LANES), jnp.int32),
)


_NW = 32
_N_IDX = B * KEEP
_PER_W = _N_IDX // _NW
_CHUNK = 64
_NCHUNK = _PER_W // _CHUNK


@functools.cache
def _make_gather():
    def body(table_hbm, idx_hbm, out_hbm, idx_v, rows0, rows1, gs0, gs1,
             os0, os1):
        wid = lax.axis_index("s") * 2 + lax.axis_index("c")
        base = wid * _PER_W
        pltpu.sync_copy(idx_hbm.at[pl.ds(base, _PER_W)], idx_v)
        rows = (rows0, rows1)
        gsem = (gs0, gs1)
        osem = (os0, os1)
        gth = [None, None]
        oth = [None, None]
        gth[0] = pltpu.async_copy(
            table_hbm.at[idx_v.at[pl.ds(0, _CHUNK)]], rows0, gs0)
        for kk in range(_NCHUNK):
            cur, nxt = kk % 2, (kk + 1) % 2
            gth[cur].wait()
            if kk >= 1:
                oth[nxt].wait()
            if kk + 1 < _NCHUNK:
                gth[nxt] = pltpu.async_copy(
                    table_hbm.at[idx_v.at[pl.ds((kk + 1) * _CHUNK, _CHUNK)]],
                    rows[nxt], gsem[nxt])
            oth[cur] = pltpu.async_copy(
                rows[cur], out_hbm.at[pl.ds(base + kk * _CHUNK, _CHUNK)],
                osem[cur])
        oth[(_NCHUNK - 1) % 2].wait()

    return pl.kernel(
        body,
        mesh=plsc.VectorSubcoreMesh(core_axis_name="c", subcore_axis_name="s"),
        out_type=jax.ShapeDtypeStruct((_N_IDX, C), jnp.float32),
        scratch_types=[
            pltpu.VMEM((_PER_W,), jnp.int32),
            pltpu.VMEM((_CHUNK, C), jnp.float32),
            pltpu.VMEM((_CHUNK, C), jnp.float32),
            pltpu.SemaphoreType.DMA,
            pltpu.SemaphoreType.DMA,
            pltpu.SemaphoreType.DMA,
            pltpu.SemaphoreType.DMA,
        ],
    )


def kernel(x, noise):
    keys = jnp.concatenate(
        [jnp.full((B, 1), jnp.inf, noise.dtype), noise], axis=1)
    gidx = _sort_call(keys.reshape(B, ROWS, LANES))
    table = x.reshape(B * T, C)
    out = _make_gather()(table, gidx.reshape(_N_IDX))
    return out.reshape(B, KEEP, C)


# speedup vs baseline: 8.5756x; 1.0357x over previous
import functools

import jax
import jax.numpy as jnp
from jax import lax
from jax.experimental import pallas as pl
from jax.experimental.pallas import tpu as pltpu
from jax.experimental.pallas import tpu_sc as plsc

B = 4
T = 8192
C = 768
KEEP = 4096
ROWS = 64
GB = 4
LANES = 128
OUT_ROWS = KEEP // LANES


def _sort_body(keys_ref, out_ref):
    vf = keys_ref[...]
    u = lax.bitcast_convert_type(vf, jnp.int32)
    v = u ^ (lax.shift_right_arithmetic(u, 31) & jnp.int32(0x7FFFFFFF))
    r = lax.broadcasted_iota(jnp.int32, (GB, ROWS, LANES), 1)
    c = lax.broadcasted_iota(jnp.int32, (GB, ROWS, LANES), 2)
    pos = r * LANES + c
    payload = pos

    for kbit in range(1, 14):
        k = 1 << kbit
        for sbit in range(kbit - 1, -1, -1):
            s = 1 << sbit
            ax, sh = (2, s) if s < LANES else (1, s // LANES)
            bit_set = (pos & s) != 0
            pv = jnp.where(bit_set, jnp.roll(v, sh, axis=ax),
                           jnp.roll(v, -sh, axis=ax))
            pi = jnp.where(bit_set, jnp.roll(payload, sh, axis=ax),
                           jnp.roll(payload, -sh, axis=ax))
            p_follows = (pv < v) | ((pv == v) & (pi > payload))
            ascending = (pos & k) == 0
            take_partner = bit_set ^ ascending ^ p_follows
            v = jnp.where(take_partner, pv, v)
            payload = jnp.where(take_partner, pi, payload)

    boff = (pl.program_id(0) * GB
            + lax.broadcasted_iota(jnp.int32, (GB, OUT_ROWS, LANES), 0)) * T
    out_ref[...] = payload[:, :OUT_ROWS, :] + boff


_sort_call = pl.pallas_call(
    _sort_body,
    grid=(B // GB,),
    in_specs=[pl.BlockSpec((GB, ROWS, LANES), lambda b: (b, 0, 0))],
    out_specs=pl.BlockSpec((GB, OUT_ROWS, LANES), lambda b: (b, 0, 0)),
    out_shape=jax.ShapeDtypeStruct((B, OUT_ROWS, LANES), jnp.int32),
)


_NW = 32
_N_IDX = B * KEEP
_PER_W = _N_IDX // _NW
_CHUNK = 32
_NCHUNK = _PER_W // _CHUNK
_NBUF = 4


@functools.cache
def _make_gather():
    def body(table_hbm, idx_hbm, out_hbm, idx_v, *bufs_and_sems):
        rows = bufs_and_sems[:_NBUF]
        gsem = bufs_and_sems[_NBUF:2 * _NBUF]
        osem = bufs_and_sems[2 * _NBUF:]
        wid = lax.axis_index("s") * 2 + lax.axis_index("c")
        base = wid * _PER_W
        pltpu.sync_copy(idx_hbm.at[pl.ds(base, _PER_W)], idx_v)

        def gather(j):
            return pltpu.async_copy(
                table_hbm.at[idx_v.at[pl.ds(j * _CHUNK, _CHUNK)]],
                rows[j % _NBUF], gsem[j % _NBUF])

        gth = [None] * _NBUF
        oth = [None] * _NBUF
        for j in range(_NBUF - 1):
            gth[j] = gather(j)
        for kk in range(_NCHUNK):
            cur = kk % _NBUF
            nj = kk + _NBUF - 1
            if nj < _NCHUNK:
                if kk >= 1:
                    oth[nj % _NBUF].wait()
                gth[nj % _NBUF] = gather(nj)
            gth[cur].wait()
            oth[cur] = pltpu.async_copy(
                rows[cur], out_hbm.at[pl.ds(base + kk * _CHUNK, _CHUNK)],
                osem[cur])
        for j in range(max(0, _NCHUNK - _NBUF), _NCHUNK):
            oth[j % _NBUF].wait()

    return pl.kernel(
        body,
        mesh=plsc.VectorSubcoreMesh(core_axis_name="c", subcore_axis_name="s"),
        out_type=jax.ShapeDtypeStruct((_N_IDX, C), jnp.float32),
        scratch_types=[
            pltpu.VMEM((_PER_W,), jnp.int32),
            *[pltpu.VMEM((_CHUNK, C), jnp.float32) for _ in range(_NBUF)],
            *[pltpu.SemaphoreType.DMA for _ in range(2 * _NBUF)],
        ],
    )


def kernel(x, noise):
    keys = jnp.concatenate(
        [jnp.full((B, 1), jnp.inf, noise.dtype), noise], axis=1)
    gidx = _sort_call(keys.reshape(B, ROWS, LANES))
    table = x.reshape(B * T, C)
    out = _make_gather()(table, gidx.reshape(_N_IDX))
    return out.reshape(B, KEEP, C)


# speedup vs baseline: 8.7792x; 1.0237x over previous
import functools

import jax
import jax.numpy as jnp
from jax import lax
from jax.experimental import pallas as pl
from jax.experimental.pallas import tpu as pltpu
from jax.experimental.pallas import tpu_sc as plsc

B = 4
T = 8192
C = 768
KEEP = 4096
ROWS = 64
GB = 4
LANES = 128
OUT_ROWS = KEEP // LANES


def _sort_body(noise_ref, out_hbm, out_vmem, sem):
    nf = noise_ref[...]
    keys2d = jnp.concatenate(
        [jnp.full((B, 1), jnp.inf, nf.dtype), nf], axis=1)
    vf = keys2d.reshape(B, ROWS, LANES)
    u = lax.bitcast_convert_type(vf, jnp.int32)
    v = u ^ (lax.shift_right_arithmetic(u, 31) & jnp.int32(0x7FFFFFFF))
    r = lax.broadcasted_iota(jnp.int32, (B, ROWS, LANES), 1)
    c = lax.broadcasted_iota(jnp.int32, (B, ROWS, LANES), 2)
    pos = r * LANES + c
    payload = pos

    for kbit in range(1, 14):
        k = 1 << kbit
        for sbit in range(kbit - 1, -1, -1):
            s = 1 << sbit
            ax, sh = (2, s) if s < LANES else (1, s // LANES)
            bit_set = (pos & s) != 0
            pv = jnp.where(bit_set, jnp.roll(v, sh, axis=ax),
                           jnp.roll(v, -sh, axis=ax))
            pi = jnp.where(bit_set, jnp.roll(payload, sh, axis=ax),
                           jnp.roll(payload, -sh, axis=ax))
            p_follows = (pv < v) | ((pv == v) & (pi > payload))
            ascending = (pos & k) == 0
            take_partner = bit_set ^ ascending ^ p_follows
            v = jnp.where(take_partner, pv, v)
            payload = jnp.where(take_partner, pi, payload)

    boff = lax.broadcasted_iota(jnp.int32, (B, OUT_ROWS, LANES), 0) * T
    out_vmem[...] = payload[:, :OUT_ROWS, :] + boff
    cp = pltpu.make_async_copy(out_vmem, out_hbm, sem)
    cp.start()
    cp.wait()


_sort_call = pl.pallas_call(
    _sort_body,
    out_specs=pl.BlockSpec(memory_space=pltpu.MemorySpace.HBM),
    out_shape=jax.ShapeDtypeStruct((B, OUT_ROWS, LANES), jnp.int32),
    scratch_shapes=[
        pltpu.VMEM((B, OUT_ROWS, LANES), jnp.int32),
        pltpu.SemaphoreType.DMA,
    ],
)


_NW = 32
_N_IDX = B * KEEP
_PER_W = _N_IDX // _NW
_CHUNK = 32
_NCHUNK = _PER_W // _CHUNK
_NBUF = 4


@functools.cache
def _make_gather():
    def body(table_hbm, idx_hbm, out_hbm, idx_v, *bufs_and_sems):
        rows = bufs_and_sems[:_NBUF]
        gsem = bufs_and_sems[_NBUF:2 * _NBUF]
        osem = bufs_and_sems[2 * _NBUF:]
        wid = lax.axis_index("s") * 2 + lax.axis_index("c")
        base = wid * _PER_W
        pltpu.sync_copy(idx_hbm.at[pl.ds(base, _PER_W)], idx_v)

        def gather(j):
            return pltpu.async_copy(
                table_hbm.at[idx_v.at[pl.ds(j * _CHUNK, _CHUNK)]],
                rows[j % _NBUF], gsem[j % _NBUF])

        gth = [None] * _NBUF
        oth = [None] * _NBUF
        for j in range(_NBUF - 1):
            gth[j] = gather(j)
        for kk in range(_NCHUNK):
            cur = kk % _NBUF
            nj = kk + _NBUF - 1
            if nj < _NCHUNK:
                if kk >= 1:
                    oth[nj % _NBUF].wait()
                gth[nj % _NBUF] = gather(nj)
            gth[cur].wait()
            oth[cur] = pltpu.async_copy(
                rows[cur], out_hbm.at[pl.ds(base + kk * _CHUNK, _CHUNK)],
                osem[cur])
        for j in range(max(0, _NCHUNK - _NBUF), _NCHUNK):
            oth[j % _NBUF].wait()

    return pl.kernel(
        body,
        mesh=plsc.VectorSubcoreMesh(core_axis_name="c", subcore_axis_name="s"),
        out_type=jax.ShapeDtypeStruct((_N_IDX, C), jnp.float32),
        scratch_types=[
            pltpu.VMEM((_PER_W,), jnp.int32),
            *[pltpu.VMEM((_CHUNK, C), jnp.float32) for _ in range(_NBUF)],
            *[pltpu.SemaphoreType.DMA for _ in range(2 * _NBUF)],
        ],
    )


def kernel(x, noise):
    gidx = _sort_call(noise)
    table = x.reshape(B * T, C)
    out = _make_gather()(table, gidx.reshape(_N_IDX))
    return out.reshape(B, KEEP, C)


# speedup vs baseline: 8.8488x; 1.0079x over previous
import functools

import jax
import jax.numpy as jnp
from jax import lax
from jax.experimental import pallas as pl
from jax.experimental.pallas import tpu as pltpu
from jax.experimental.pallas import tpu_sc as plsc

B = 4
T = 8192
C = 768
KEEP = 4096
ROWS = 64
GB = 4
LANES = 128
OUT_ROWS = KEEP // LANES


def _sort_body(noise_ref, out_hbm, out_vmem, sem):
    nf = noise_ref[...]
    keys2d = jnp.concatenate(
        [jnp.full((B, 1), jnp.inf, nf.dtype), nf], axis=1)
    vf = keys2d.reshape(B, ROWS, LANES)
    u = lax.bitcast_convert_type(vf, jnp.int32)
    v = u ^ (lax.shift_right_arithmetic(u, 31) & jnp.int32(0x7FFFFFFF))
    r = lax.broadcasted_iota(jnp.int32, (B, ROWS, LANES), 1)
    c = lax.broadcasted_iota(jnp.int32, (B, ROWS, LANES), 2)
    payload = r * LANES + c
    c1 = lax.broadcasted_iota(jnp.int32, (1, 1, LANES), 2)
    r1 = lax.broadcasted_iota(jnp.int32, (1, ROWS, 1), 1)

    def posbit(m):
        return (c1 & m) != 0 if m < LANES else (r1 & (m // LANES)) != 0

    for kbit in range(1, 14):
        k = 1 << kbit
        for sbit in range(kbit - 1, -1, -1):
            s = 1 << sbit
            ax, sh = (2, s) if s < LANES else (1, s // LANES)
            bit_set = posbit(s)
            m = (bit_set ^ ~posbit(k)) if k < T else ~bit_set
            pv = jnp.where(bit_set, jnp.roll(v, sh, axis=ax),
                           jnp.roll(v, -sh, axis=ax))
            pi = jnp.where(bit_set, jnp.roll(payload, sh, axis=ax),
                           jnp.roll(payload, -sh, axis=ax))
            p_follows = (pv < v) | ((pv == v) & (pi > payload))
            take_partner = m ^ p_follows
            v = jnp.where(take_partner, pv, v)
            payload = jnp.where(take_partner, pi, payload)

    boff = lax.broadcasted_iota(jnp.int32, (B, OUT_ROWS, LANES), 0) * T
    out_vmem[...] = payload[:, :OUT_ROWS, :] + boff
    cp = pltpu.make_async_copy(out_vmem, out_hbm, sem)
    cp.start()
    cp.wait()


_sort_call = pl.pallas_call(
    _sort_body,
    out_specs=pl.BlockSpec(memory_space=pltpu.MemorySpace.HBM),
    out_shape=jax.ShapeDtypeStruct((B, OUT_ROWS, LANES), jnp.int32),
    scratch_shapes=[
        pltpu.VMEM((B, OUT_ROWS, LANES), jnp.int32),
        pltpu.SemaphoreType.DMA,
    ],
)


_NW = 32
_N_IDX = B * KEEP
_PER_W = _N_IDX // _NW
_CHUNK = 32
_NCHUNK = _PER_W // _CHUNK
_NBUF = 4


@functools.cache
def _make_gather():
    def body(table_hbm, idx_hbm, out_hbm, idx_v, *bufs_and_sems):
        rows = bufs_and_sems[:_NBUF]
        gsem = bufs_and_sems[_NBUF:2 * _NBUF]
        osem = bufs_and_sems[2 * _NBUF:]
        wid = lax.axis_index("s") * 2 + lax.axis_index("c")
        base = wid * _PER_W
        pltpu.sync_copy(idx_hbm.at[pl.ds(base, _PER_W)], idx_v)

        def gather(j):
            return pltpu.async_copy(
                table_hbm.at[idx_v.at[pl.ds(j * _CHUNK, _CHUNK)]],
                rows[j % _NBUF], gsem[j % _NBUF])

        gth = [None] * _NBUF
        oth = [None] * _NBUF
        for j in range(_NBUF - 1):
            gth[j] = gather(j)
        for kk in range(_NCHUNK):
            cur = kk % _NBUF
            nj = kk + _NBUF - 1
            if nj < _NCHUNK:
                if kk >= 1:
                    oth[nj % _NBUF].wait()
                gth[nj % _NBUF] = gather(nj)
            gth[cur].wait()
            oth[cur] = pltpu.async_copy(
                rows[cur], out_hbm.at[pl.ds(base + kk * _CHUNK, _CHUNK)],
                osem[cur])
        for j in range(max(0, _NCHUNK - _NBUF), _NCHUNK):
            oth[j % _NBUF].wait()

    return pl.kernel(
        body,
        mesh=plsc.VectorSubcoreMesh(core_axis_name="c", subcore_axis_name="s"),
        out_type=jax.ShapeDtypeStruct((_N_IDX, C), jnp.float32),
        scratch_types=[
            pltpu.VMEM((_PER_W,), jnp.int32),
            *[pltpu.VMEM((_CHUNK, C), jnp.float32) for _ in range(_NBUF)],
            *[pltpu.SemaphoreType.DMA for _ in range(2 * _NBUF)],
        ],
    )


def kernel(x, noise):
    gidx = _sort_call(noise)
    table = x.reshape(B * T, C)
    out = _make_gather()(table, gidx.reshape(_N_IDX))
    return out.reshape(B, KEEP, C)


# speedup vs baseline: 8.8617x; 1.0015x over previous
import functools

import jax
import jax.numpy as jnp
from jax import lax
from jax.experimental import pallas as pl
from jax.experimental.pallas import tpu as pltpu
from jax.experimental.pallas import tpu_sc as plsc

B = 4
T = 8192
C = 768
KEEP = 4096
ROWS = 64
GB = 4
LANES = 128
OUT_ROWS = KEEP // LANES


def _sort_body(noise_ref, out_hbm, out_vmem, sem):
    nf = noise_ref[...]
    keys2d = jnp.concatenate(
        [jnp.full((B, 1), jnp.inf, nf.dtype), nf], axis=1)
    vf = keys2d.reshape(B, ROWS, LANES)
    u = lax.bitcast_convert_type(vf, jnp.int32)
    v = u ^ (lax.shift_right_arithmetic(u, 31) & jnp.int32(0x7FFFFFFF))
    r = lax.broadcasted_iota(jnp.int32, (B, ROWS, LANES), 1)
    c = lax.broadcasted_iota(jnp.int32, (B, ROWS, LANES), 2)
    payload = r * LANES + c
    c1 = lax.broadcasted_iota(jnp.int32, (1, 1, LANES), 2)
    r1 = lax.broadcasted_iota(jnp.int32, (1, ROWS, 1), 1)

    def posbit(m):
        return (c1 & m) != 0 if m < LANES else (r1 & (m // LANES)) != 0

    for kbit in range(1, 14):
        k = 1 << kbit
        for sbit in range(kbit - 1, -1, -1):
            s = 1 << sbit
            ax, sh = (2, s) if s < LANES else (1, s // LANES)
            if k == T and s == T // 4:
                v = v[:, : ROWS // 2, :]
                payload = payload[:, : ROWS // 2, :]
                r1 = r1[:, : ROWS // 2, :]
            bit_set = posbit(s)
            m = (bit_set ^ ~posbit(k)) if k < T else ~bit_set
            pv = jnp.where(bit_set, jnp.roll(v, sh, axis=ax),
                           jnp.roll(v, -sh, axis=ax))
            pi = jnp.where(bit_set, jnp.roll(payload, sh, axis=ax),
                           jnp.roll(payload, -sh, axis=ax))
            p_follows = (pv < v) | ((pv == v) & (pi > payload))
            take_partner = m ^ p_follows
            v = jnp.where(take_partner, pv, v)
            payload = jnp.where(take_partner, pi, payload)

    boff = lax.broadcasted_iota(jnp.int32, (B, OUT_ROWS, LANES), 0) * T
    out_vmem[...] = payload + boff
    cp = pltpu.make_async_copy(out_vmem, out_hbm, sem)
    cp.start()
    cp.wait()


_sort_call = pl.pallas_call(
    _sort_body,
    out_specs=pl.BlockSpec(memory_space=pltpu.MemorySpace.HBM),
    out_shape=jax.ShapeDtypeStruct((B, OUT_ROWS, LANES), jnp.int32),
    scratch_shapes=[
        pltpu.VMEM((B, OUT_ROWS, LANES), jnp.int32),
        pltpu.SemaphoreType.DMA,
    ],
)


_NW = 32
_N_IDX = B * KEEP
_PER_W = _N_IDX // _NW
_CHUNK = 32
_NCHUNK = _PER_W // _CHUNK
_NBUF = 4


@functools.cache
def _make_gather():
    def body(table_hbm, idx_hbm, out_hbm, idx_v, *bufs_and_sems):
        rows = bufs_and_sems[:_NBUF]
        gsem = bufs_and_sems[_NBUF:2 * _NBUF]
        osem = bufs_and_sems[2 * _NBUF:]
        wid = lax.axis_index("s") * 2 + lax.axis_index("c")
        base = wid * _PER_W
        pltpu.sync_copy(idx_hbm.at[pl.ds(base, _PER_W)], idx_v)

        def gather(j):
            return pltpu.async_copy(
                table_hbm.at[idx_v.at[pl.ds(j * _CHUNK, _CHUNK)]],
                rows[j % _NBUF], gsem[j % _NBUF])

        gth = [None] * _NBUF
        oth = [None] * _NBUF
        for j in range(_NBUF - 1):
            gth[j] = gather(j)
        for kk in range(_NCHUNK):
            cur = kk % _NBUF
            nj = kk + _NBUF - 1
            if nj < _NCHUNK:
                if kk >= 1:
                    oth[nj % _NBUF].wait()
                gth[nj % _NBUF] = gather(nj)
            gth[cur].wait()
            oth[cur] = pltpu.async_copy(
                rows[cur], out_hbm.at[pl.ds(base + kk * _CHUNK, _CHUNK)],
                osem[cur])
        for j in range(max(0, _NCHUNK - _NBUF), _NCHUNK):
            oth[j % _NBUF].wait()

    return pl.kernel(
        body,
        mesh=plsc.VectorSubcoreMesh(core_axis_name="c", subcore_axis_name="s"),
        out_type=jax.ShapeDtypeStruct((_N_IDX, C), jnp.float32),
        scratch_types=[
            pltpu.VMEM((_PER_W,), jnp.int32),
            *[pltpu.VMEM((_CHUNK, C), jnp.float32) for _ in range(_NBUF)],
            *[pltpu.SemaphoreType.DMA for _ in range(2 * _NBUF)],
        ],
    )


def kernel(x, noise):
    gidx = _sort_call(noise)
    table = x.reshape(B * T, C)
    out = _make_gather()(table, gidx.reshape(_N_IDX))
    return out.reshape(B, KEEP, C)


# speedup vs baseline: 8.9363x; 1.0084x over previous
import functools

import jax
import jax.numpy as jnp
from jax import lax
from jax.experimental import pallas as pl
from jax.experimental.pallas import tpu as pltpu
from jax.experimental.pallas import tpu_sc as plsc

B = 4
T = 8192
C = 768
KEEP = 4096
ROWS = 64
GB = 4
LANES = 128
OUT_ROWS = KEEP // LANES


def _sort_body(noise_ref, out_hbm, out_vmem, sem):
    nf = noise_ref[...]
    keys2d = jnp.concatenate(
        [jnp.full((B, 1), jnp.inf, nf.dtype), nf], axis=1)
    vf = keys2d.reshape(B, ROWS, LANES)
    u = lax.bitcast_convert_type(vf, jnp.int32)
    v = u ^ (lax.shift_right_arithmetic(u, 31) & jnp.int32(0x7FFFFFFF))
    r = lax.broadcasted_iota(jnp.int32, (B, ROWS, LANES), 1)
    c = lax.broadcasted_iota(jnp.int32, (B, ROWS, LANES), 2)
    payload = r * LANES + c
    c1 = lax.broadcasted_iota(jnp.int32, (1, 1, LANES), 2)
    r1 = lax.broadcasted_iota(jnp.int32, (1, ROWS, 1), 1)

    def posbit(m):
        return (c1 & m) != 0 if m < LANES else (r1 & (m // LANES)) != 0

    for kbit in range(1, 14):
        k = 1 << kbit
        for sbit in range(kbit - 1, -1, -1):
            s = 1 << sbit
            ax, sh = (2, s) if s < LANES else (1, s // LANES)
            if k == T and s == T // 4:
                v = v[:, : ROWS // 2, :]
                payload = payload[:, : ROWS // 2, :]
                r1 = r1[:, : ROWS // 2, :]
            bit_set = posbit(s)
            m = (bit_set ^ ~posbit(k)) if k < T else ~bit_set
            pv = jnp.where(bit_set, jnp.roll(v, sh, axis=ax),
                           jnp.roll(v, -sh, axis=ax))
            pi = jnp.where(bit_set, jnp.roll(payload, sh, axis=ax),
                           jnp.roll(payload, -sh, axis=ax))
            p_follows = (pv < v) | ((pv == v) & (pi > payload))
            take_partner = m ^ p_follows
            v = jnp.where(take_partner, pv, v)
            payload = jnp.where(take_partner, pi, payload)

    boff = lax.broadcasted_iota(jnp.int32, (B, OUT_ROWS, LANES), 0) * T
    out_vmem[...] = payload + boff
    cp = pltpu.make_async_copy(out_vmem, out_hbm, sem)
    cp.start()
    cp.wait()


_sort_call = pl.pallas_call(
    _sort_body,
    out_specs=pl.BlockSpec(memory_space=pltpu.MemorySpace.HBM),
    out_shape=jax.ShapeDtypeStruct((B, OUT_ROWS, LANES), jnp.int32),
    scratch_shapes=[
        pltpu.VMEM((B, OUT_ROWS, LANES), jnp.int32),
        pltpu.SemaphoreType.DMA,
    ],
)


_NW = 32
_N_IDX = B * KEEP
_PER_W = _N_IDX // _NW
_CHUNK = 32
_NCHUNK = _PER_W // _CHUNK
_NBUF = 5


@functools.cache
def _make_gather():
    def body(table_hbm, idx_hbm, out_hbm, idx_v, *bufs_and_sems):
        rows = bufs_and_sems[:_NBUF]
        gsem = bufs_and_sems[_NBUF:2 * _NBUF]
        osem = bufs_and_sems[2 * _NBUF:]
        wid = lax.axis_index("s") * 2 + lax.axis_index("c")
        base = wid * _PER_W
        pltpu.sync_copy(idx_hbm.at[pl.ds(base, _PER_W)], idx_v)

        def gather(j):
            return pltpu.async_copy(
                table_hbm.at[idx_v.at[pl.ds(j * _CHUNK, _CHUNK)]],
                rows[j % _NBUF], gsem[j % _NBUF])

        gth = [None] * _NBUF
        oth = [None] * _NBUF
        for j in range(_NBUF - 1):
            gth[j] = gather(j)
        for kk in range(_NCHUNK):
            cur = kk % _NBUF
            nj = kk + _NBUF - 1
            if nj < _NCHUNK:
                if kk >= 1:
                    oth[nj % _NBUF].wait()
                gth[nj % _NBUF] = gather(nj)
            gth[cur].wait()
            oth[cur] = pltpu.async_copy(
                rows[cur], out_hbm.at[pl.ds(base + kk * _CHUNK, _CHUNK)],
                osem[cur])
        for j in range(max(0, _NCHUNK - _NBUF), _NCHUNK):
            oth[j % _NBUF].wait()

    return pl.kernel(
        body,
        mesh=plsc.VectorSubcoreMesh(core_axis_name="c", subcore_axis_name="s"),
        out_type=jax.ShapeDtypeStruct((_N_IDX, C), jnp.float32),
        scratch_types=[
            pltpu.VMEM((_PER_W,), jnp.int32),
            *[pltpu.VMEM((_CHUNK, C), jnp.float32) for _ in range(_NBUF)],
            *[pltpu.SemaphoreType.DMA for _ in range(2 * _NBUF)],
        ],
    )


def kernel(x, noise):
    gidx = _sort_call(noise)
    table = x.reshape(B * T, C)
    out = _make_gather()(table, gidx.reshape(_N_IDX))
    return out.reshape(B, KEEP, C)
